# baseline (speedup 1.0000x reference)
import functools

import jax
import jax.numpy as jnp
from jax import lax
from jax.experimental import pallas as pl
from jax.experimental.pallas import tpu as pltpu
from jax.experimental.pallas import tpu_sc as plsc

D_MODEL = 1024
SEQ = 4096

_R_TC = 2048
_SC_ROWS = SEQ - _R_TC

_info = plsc.get_sparse_core_info()
_NC, _NS = _info.num_cores, _info.num_subcores
_NW = _NC * _NS
_ROWS_PER_W = _SC_ROWS // _NW
_CHUNK = 16
_NBUF = 4
_NCHUNK = _ROWS_PER_W // _CHUNK

_mesh = plsc.VectorSubcoreMesh(core_axis_name="c", subcore_axis_name="s")


@functools.partial(
    pl.kernel,
    mesh=_mesh,
    out_type=jax.ShapeDtypeStruct((SEQ, D_MODEL), jnp.float32),
    scratch_types=(
        [pltpu.VMEM((_CHUNK, D_MODEL), jnp.float32) for _ in range(_NBUF)]
        + [pltpu.SemaphoreType.DMA for _ in range(2 * _NBUF)]
    ),
)
def _pe_slice_copy_sc(pe_hbm, out_hbm, *scratch):
    bufs = scratch[:_NBUF]
    sins = scratch[_NBUF : 2 * _NBUF]
    souts = scratch[2 * _NBUF :]
    wid = lax.axis_index("s") * _NC + lax.axis_index("c")
    base = _R_TC + wid * _ROWS_PER_W

    in_h = [None] * _NBUF
    out_h = [None] * _NBUF
    nprime = min(_NBUF, _NCHUNK)
    for j in range(nprime):
        in_h[j] = pltpu.async_copy(
            pe_hbm.at[pl.ds(base + j * _CHUNK, _CHUNK)], bufs[j], sins[j]
        )
    for i in range(_NCHUNK):
        j = i % _NBUF
        in_h[j].wait()
        out_h[j] = pltpu.async_copy(
            bufs[j], out_hbm.at[pl.ds(base + i * _CHUNK, _CHUNK)], souts[j]
        )
        nxt = i + _NBUF
        if nxt < _NCHUNK:
            out_h[j].wait()
            in_h[j] = pltpu.async_copy(
                pe_hbm.at[pl.ds(base + nxt * _CHUNK, _CHUNK)], bufs[j], sins[j]
            )
    for j in range(nprime):
        out_h[(_NCHUNK - 1 - j) % _NBUF].wait()


_TC_BLK = 256


def _tc_body(pe_ref, alias_ref, out_ref):
    del alias_ref
    out_ref[...] = pe_ref[...]


_pe_slice_copy_tc = pl.pallas_call(
    _tc_body,
    grid=(_R_TC // _TC_BLK,),
    in_specs=[
        pl.BlockSpec((_TC_BLK, D_MODEL), lambda i: (i, 0)),
        pl.BlockSpec(memory_space=pl.ANY),
    ],
    out_specs=pl.BlockSpec((_TC_BLK, D_MODEL), lambda i: (i, 0)),
    out_shape=jax.ShapeDtypeStruct((SEQ, D_MODEL), jnp.float32),
    input_output_aliases={1: 0},
)


def kernel(x, pe):
    del x
    table = pe[0]
    sc_out = _pe_slice_copy_sc(table)
    out = _pe_slice_copy_tc(table, sc_out)
    return out[None]


# speedup vs baseline: 1.1544x; 1.1544x over previous
import functools

import jax
import jax.numpy as jnp
from jax import lax
from jax.experimental import pallas as pl
from jax.experimental.pallas import tpu as pltpu
from jax.experimental.pallas import tpu_sc as plsc

D_MODEL = 1024
SEQ = 4096

_info = plsc.get_sparse_core_info()
_NC, _NS = _info.num_cores, _info.num_subcores
_NW = _NC * _NS
_ROWS_PER_W = SEQ // _NW
_CHUNK = 16
_NBUF = 7
_NCHUNK = _ROWS_PER_W // _CHUNK

_mesh = plsc.VectorSubcoreMesh(core_axis_name="c", subcore_axis_name="s")


@functools.partial(
    pl.kernel,
    mesh=_mesh,
    out_type=jax.ShapeDtypeStruct((SEQ, D_MODEL), jnp.float32),
    scratch_types=(
        [pltpu.VMEM((_CHUNK, D_MODEL), jnp.float32) for _ in range(_NBUF)]
        + [pltpu.SemaphoreType.DMA for _ in range(2 * _NBUF)]
    ),
)
def _pe_slice_copy(pe_hbm, out_hbm, *scratch):
    bufs = scratch[:_NBUF]
    sins = scratch[_NBUF : 2 * _NBUF]
    souts = scratch[2 * _NBUF :]
    wid = lax.axis_index("s") * _NC + lax.axis_index("c")
    base = wid * _ROWS_PER_W

    in_h = [None] * _NBUF
    out_h = [None] * _NBUF
    for j in range(_NBUF):
        in_h[j] = pltpu.async_copy(
            pe_hbm.at[pl.ds(base + j * _CHUNK, _CHUNK)], bufs[j], sins[j]
        )
    for i in range(_NCHUNK):
        j = i % _NBUF
        in_h[j].wait()
        out_h[j] = pltpu.async_copy(
            bufs[j], out_hbm.at[pl.ds(base + i * _CHUNK, _CHUNK)], souts[j]
        )
        nxt = i + _NBUF
        if nxt < _NCHUNK:
            out_h[j].wait()
            in_h[j] = pltpu.async_copy(
                pe_hbm.at[pl.ds(base + nxt * _CHUNK, _CHUNK)], bufs[j], sins[j]
            )
    for j in range(_NBUF):
        if out_h[j] is not None:
            out_h[j].wait()


def kernel(x, pe):
    del x
    return _pe_slice_copy(pe[0])[None]
